# Initial kernel scaffold; baseline (speedup 1.0000x reference)
import jax
import jax.numpy as jnp
from jax.experimental import pallas as pl


def kernel(inputs, Wq, bq, Wk, bk, Wv, bv, Wo, bo, g1, beta1, g2, beta2, W1, bf1, W2, bf2):
    raise NotImplementedError("write your pallas kernel here")


# speedup vs baseline: 1.0454x; 1.0454x over previous
import functools

import jax
import jax.numpy as jnp
import numpy as np
from jax.experimental import pallas as pl

_S = 2048
_BS = 64
_NB = _S // _BS
_H = 12
_DH = 64
_D = 768
_FF = 3072
_NR = 3


def _block_mask() -> np.ndarray:
    m = np.zeros((_NB, _NB), dtype=bool)
    idx = np.arange(_NB)
    for off in (-1, 0, 1):
        j = idx + off
        ok = (j >= 0) & (j < _NB)
        m[idx[ok], j[ok]] = True
    m[0, :] = True
    m[-1, :] = True
    m[:, 0] = True
    m[:, -1] = True
    rng = np.random.RandomState(0)
    for i in range(1, _NB - 1):
        cand = np.array([j for j in range(1, _NB - 1) if abs(j - i) > 1])
        sel = rng.choice(cand, size=_NR, replace=False)
        m[i, sel] = True
    return m


_BM = _block_mask()
_MAXC = 8
_COLS = {}
_NVAL = {}
for _i in range(1, _NB - 1):
    _c = list(np.where(_BM[_i])[0])
    _NVAL[_i] = len(_c)
    while len(_c) < _MAXC:
        _c.append(_c[-1])
    _COLS[_i] = _c


def _qkv_kernel(x_ref, w_ref, b_ref, o_ref):
    o_ref[...] = (
        jnp.dot(x_ref[...], w_ref[...], preferred_element_type=jnp.float32)
        + b_ref[...]
    )


def _attn_kernel(q_ref, k_ref, v_ref, o_ref):
    k = k_ref[0]
    v = v_ref[0]
    outs = []
    for i in range(_NB):
        q = q_ref[0, i * _BS:(i + 1) * _BS, :] * 0.125
        if i == 0 or i == _NB - 1:
            s = jax.lax.dot_general(
                q, k, (((1,), (1,)), ((), ())),
                preferred_element_type=jnp.float32)
            p = jnp.exp(s - jnp.max(s, axis=1, keepdims=True))
            ctx = jax.lax.dot_general(
                p / jnp.sum(p, axis=1, keepdims=True), v,
                (((1,), (0,)), ((), ())),
                preferred_element_type=jnp.float32)
        else:
            cols = _COLS[i]
            nval = _NVAL[i]
            kg = jnp.concatenate(
                [k[c * _BS:(c + 1) * _BS, :] for c in cols], axis=0)
            vg = jnp.concatenate(
                [v[c * _BS:(c + 1) * _BS, :] for c in cols], axis=0)
            s = jax.lax.dot_general(
                q, kg, (((1,), (1,)), ((), ())),
                preferred_element_type=jnp.float32)
            if nval < _MAXC:
                lane = jax.lax.broadcasted_iota(
                    jnp.int32, (_BS, _MAXC * _BS), 1)
                s = jnp.where(lane < nval * _BS, s, jnp.float32(-1e9))
            p = jnp.exp(s - jnp.max(s, axis=1, keepdims=True))
            ctx = jax.lax.dot_general(
                p / jnp.sum(p, axis=1, keepdims=True), vg,
                (((1,), (0,)), ((), ())),
                preferred_element_type=jnp.float32)
        outs.append(ctx)
    o_ref[0] = jnp.concatenate(outs, axis=0)


def _ln(t, g, b):
    mu = jnp.mean(t, axis=1, keepdims=True)
    var = jnp.mean((t - mu) ** 2, axis=1, keepdims=True)
    return (t - mu) / jnp.sqrt(var + 1e-6) * g + b


def _ffn_kernel(x_ref, c_ref, wo_ref, bo_ref, g1_ref, b1_ref, w1_ref,
                bf1_ref, w2_ref, bf2_ref, g2_ref, b2_ref, o_ref):
    attn = (
        jnp.dot(c_ref[...], wo_ref[...], preferred_element_type=jnp.float32)
        + bo_ref[...]
    )
    out1 = _ln(x_ref[...] + attn, g1_ref[...], b1_ref[...])
    h = jnp.maximum(
        jnp.dot(out1, w1_ref[...], preferred_element_type=jnp.float32)
        + bf1_ref[...], 0.0)
    ffn = (
        jnp.dot(h, w2_ref[...], preferred_element_type=jnp.float32)
        + bf2_ref[...]
    )
    o_ref[...] = _ln(out1 + ffn, g2_ref[...], b2_ref[...])


_ROWC = 256
_NRC = _S // _ROWC


@functools.partial(jax.jit, static_argnums=())
def kernel(inputs, Wq, bq, Wk, bk, Wv, bv, Wo, bo, g1, beta1, g2, beta2,
           W1, bf1, W2, bf2):
    x = inputs[0]

    wqkv = jnp.concatenate(
        [Wq.reshape(_D, _H * _DH), Wk.reshape(_D, _H * _DH),
         Wv.reshape(_D, _H * _DH)], axis=1)
    bqkv = jnp.concatenate(
        [bq.reshape(1, _H * _DH), bk.reshape(1, _H * _DH),
         bv.reshape(1, _H * _DH)], axis=1)

    qkv = pl.pallas_call(
        _qkv_kernel,
        grid=(_NRC,),
        in_specs=[
            pl.BlockSpec((_ROWC, _D), lambda r: (r, 0)),
            pl.BlockSpec((_D, 3 * _H * _DH), lambda r: (0, 0)),
            pl.BlockSpec((1, 3 * _H * _DH), lambda r: (0, 0)),
        ],
        out_specs=pl.BlockSpec((_ROWC, 3 * _H * _DH), lambda r: (r, 0)),
        out_shape=jax.ShapeDtypeStruct((_S, 3 * _H * _DH), jnp.float32),
    )(x, wqkv, bqkv)

    def heads(a):
        return a.reshape(_S, _H, _DH).transpose(1, 0, 2)

    q3 = heads(qkv[:, : _H * _DH])
    k3 = heads(qkv[:, _H * _DH: 2 * _H * _DH])
    v3 = heads(qkv[:, 2 * _H * _DH:])

    ctx = pl.pallas_call(
        _attn_kernel,
        grid=(_H,),
        in_specs=[
            pl.BlockSpec((1, _S, _DH), lambda h: (h, 0, 0)),
            pl.BlockSpec((1, _S, _DH), lambda h: (h, 0, 0)),
            pl.BlockSpec((1, _S, _DH), lambda h: (h, 0, 0)),
        ],
        out_specs=pl.BlockSpec((1, _S, _DH), lambda h: (h, 0, 0)),
        out_shape=jax.ShapeDtypeStruct((_H, _S, _DH), jnp.float32),
    )(q3, k3, v3)

    ctx2 = ctx.transpose(1, 0, 2).reshape(_S, _H * _DH)

    out = pl.pallas_call(
        _ffn_kernel,
        grid=(_NRC,),
        in_specs=[
            pl.BlockSpec((_ROWC, _D), lambda r: (r, 0)),
            pl.BlockSpec((_ROWC, _H * _DH), lambda r: (r, 0)),
            pl.BlockSpec((_H * _DH, _D), lambda r: (0, 0)),
            pl.BlockSpec((1, _D), lambda r: (0, 0)),
            pl.BlockSpec((1, _D), lambda r: (0, 0)),
            pl.BlockSpec((1, _D), lambda r: (0, 0)),
            pl.BlockSpec((_D, _FF), lambda r: (0, 0)),
            pl.BlockSpec((1, _FF), lambda r: (0, 0)),
            pl.BlockSpec((_FF, _D), lambda r: (0, 0)),
            pl.BlockSpec((1, _D), lambda r: (0, 0)),
            pl.BlockSpec((1, _D), lambda r: (0, 0)),
            pl.BlockSpec((1, _D), lambda r: (0, 0)),
        ],
        out_specs=pl.BlockSpec((_ROWC, _D), lambda r: (r, 0)),
        out_shape=jax.ShapeDtypeStruct((_S, _D), jnp.float32),
    )(x, ctx2, Wo.reshape(_H * _DH, _D), bo.reshape(1, _D),
      g1.reshape(1, _D), beta1.reshape(1, _D), W1, bf1.reshape(1, _FF),
      W2, bf2.reshape(1, _D), g2.reshape(1, _D), beta2.reshape(1, _D))

    return out[None]


# speedup vs baseline: 1.2300x; 1.1766x over previous
import functools

import jax
import jax.numpy as jnp
import numpy as np
from jax.experimental import pallas as pl

_S = 2048
_BS = 64
_NB = _S // _BS
_H = 12
_DH = 64
_D = 768
_FF = 3072
_NR = 3
_G = 3
_HG = _H // _G
_GW = _HG * _DH
_MAXC = 8


def _block_mask() -> np.ndarray:
    m = np.zeros((_NB, _NB), dtype=bool)
    idx = np.arange(_NB)
    for off in (-1, 0, 1):
        j = idx + off
        ok = (j >= 0) & (j < _NB)
        m[idx[ok], j[ok]] = True
    m[0, :] = True
    m[-1, :] = True
    m[:, 0] = True
    m[:, -1] = True
    rng = np.random.RandomState(0)
    for i in range(1, _NB - 1):
        cand = np.array([j for j in range(1, _NB - 1) if abs(j - i) > 1])
        sel = rng.choice(cand, size=_NR, replace=False)
        m[i, sel] = True
    return m


_BM = _block_mask()
_COLS = {}
_NVAL = {}
for _i in range(1, _NB - 1):
    _c = list(np.where(_BM[_i])[0])
    _NVAL[_i] = len(_c)
    while len(_c) < _MAXC:
        _c.append(_c[-1])
    _COLS[_i] = _c


def _attn_kernel(x_ref, wq_ref, wk_ref, wv_ref, bq_ref, bk_ref, bv_ref,
                 o_ref):
    x = x_ref[...]
    qf = (jnp.dot(x, wq_ref[...], preferred_element_type=jnp.float32)
          + bq_ref[...])
    kf = (jnp.dot(x, wk_ref[...], preferred_element_type=jnp.float32)
          + bk_ref[...])
    vf = (jnp.dot(x, wv_ref[...], preferred_element_type=jnp.float32)
          + bv_ref[...])
    kb = kf.astype(jnp.bfloat16)
    vb = vf.astype(jnp.bfloat16)

    lane_g = jax.lax.broadcasted_iota(jnp.int32, (_S, _GW), 1) // _DH
    qm = [(qf * jnp.where(lane_g == h, jnp.float32(0.125),
                          jnp.float32(0.0))).astype(jnp.bfloat16)
          for h in range(_HG)]
    out_lane = jax.lax.broadcasted_iota(jnp.int32, (_BS, _GW), 1) // _DH

    outs = []
    for i in range(_NB):
        if i == 0 or i == _NB - 1:
            kg, vg = kb, vb
            nval = _NB
        else:
            cols = _COLS[i]
            nval = _NVAL[i]
            kg = jnp.concatenate(
                [kb[c * _BS:(c + 1) * _BS, :] for c in cols], axis=0)
            vg = jnp.concatenate(
                [vb[c * _BS:(c + 1) * _BS, :] for c in cols], axis=0)
        nk = kg.shape[0]
        acc = jnp.zeros((_BS, _GW), jnp.float32)
        for h in range(_HG):
            q_i = qm[h][i * _BS:(i + 1) * _BS, :]
            s = jax.lax.dot_general(
                q_i, kg, (((1,), (1,)), ((), ())),
                preferred_element_type=jnp.float32)
            if nval * _BS < nk:
                lane = jax.lax.broadcasted_iota(jnp.int32, (_BS, nk), 1)
                s = jnp.where(lane < nval * _BS, s, jnp.float32(-1e9))
            m = jnp.max(s, axis=1, keepdims=True)
            e = jnp.exp(s - m)
            p = (e / jnp.sum(e, axis=1, keepdims=True)).astype(jnp.bfloat16)
            ctx = jax.lax.dot_general(
                p, vg, (((1,), (0,)), ((), ())),
                preferred_element_type=jnp.float32)
            acc = acc + jnp.where(out_lane == h, ctx, 0.0)
        outs.append(acc.astype(jnp.bfloat16))
    o_ref[...] = jnp.concatenate(outs, axis=0)


def _ln(t, g, b):
    mu = jnp.mean(t, axis=1, keepdims=True)
    var = jnp.mean((t - mu) ** 2, axis=1, keepdims=True)
    return (t - mu) / jnp.sqrt(var + 1e-6) * g + b


def _ffn_kernel(x_ref, c_ref, wo_ref, bo_ref, g1_ref, b1_ref, w1_ref,
                bf1_ref, w2_ref, bf2_ref, g2_ref, b2_ref, o_ref):
    attn = (
        jnp.dot(c_ref[...], wo_ref[...], preferred_element_type=jnp.float32)
        + bo_ref[...]
    )
    out1 = _ln(x_ref[...] + attn, g1_ref[...], b1_ref[...])
    h = jnp.maximum(
        jnp.dot(out1.astype(jnp.bfloat16), w1_ref[...],
                preferred_element_type=jnp.float32)
        + bf1_ref[...], 0.0)
    ffn = (
        jnp.dot(h.astype(jnp.bfloat16), w2_ref[...],
                preferred_element_type=jnp.float32)
        + bf2_ref[...]
    )
    o_ref[...] = _ln(out1 + ffn, g2_ref[...], b2_ref[...])


_ROWC = 256
_NRC = _S // _ROWC


@functools.partial(jax.jit, static_argnums=())
def kernel(inputs, Wq, bq, Wk, bk, Wv, bv, Wo, bo, g1, beta1, g2, beta2,
           W1, bf1, W2, bf2):
    x = inputs[0]
    xb = x.astype(jnp.bfloat16)
    wq = Wq.reshape(_D, _H * _DH).astype(jnp.bfloat16)
    wk = Wk.reshape(_D, _H * _DH).astype(jnp.bfloat16)
    wv = Wv.reshape(_D, _H * _DH).astype(jnp.bfloat16)
    bq2 = bq.reshape(1, _H * _DH)
    bk2 = bk.reshape(1, _H * _DH)
    bv2 = bv.reshape(1, _H * _DH)

    ctx = pl.pallas_call(
        _attn_kernel,
        grid=(_G,),
        in_specs=[
            pl.BlockSpec((_S, _D), lambda g: (0, 0)),
            pl.BlockSpec((_D, _GW), lambda g: (0, g)),
            pl.BlockSpec((_D, _GW), lambda g: (0, g)),
            pl.BlockSpec((_D, _GW), lambda g: (0, g)),
            pl.BlockSpec((1, _GW), lambda g: (0, g)),
            pl.BlockSpec((1, _GW), lambda g: (0, g)),
            pl.BlockSpec((1, _GW), lambda g: (0, g)),
        ],
        out_specs=pl.BlockSpec((_S, _GW), lambda g: (0, g)),
        out_shape=jax.ShapeDtypeStruct((_S, _H * _DH), jnp.bfloat16),
    )(xb, wq, wk, wv, bq2, bk2, bv2)

    out = pl.pallas_call(
        _ffn_kernel,
        grid=(_NRC,),
        in_specs=[
            pl.BlockSpec((_ROWC, _D), lambda r: (r, 0)),
            pl.BlockSpec((_ROWC, _H * _DH), lambda r: (r, 0)),
            pl.BlockSpec((_H * _DH, _D), lambda r: (0, 0)),
            pl.BlockSpec((1, _D), lambda r: (0, 0)),
            pl.BlockSpec((1, _D), lambda r: (0, 0)),
            pl.BlockSpec((1, _D), lambda r: (0, 0)),
            pl.BlockSpec((_D, _FF), lambda r: (0, 0)),
            pl.BlockSpec((1, _FF), lambda r: (0, 0)),
            pl.BlockSpec((_FF, _D), lambda r: (0, 0)),
            pl.BlockSpec((1, _D), lambda r: (0, 0)),
            pl.BlockSpec((1, _D), lambda r: (0, 0)),
            pl.BlockSpec((1, _D), lambda r: (0, 0)),
        ],
        out_specs=pl.BlockSpec((_ROWC, _D), lambda r: (r, 0)),
        out_shape=jax.ShapeDtypeStruct((_S, _D), jnp.float32),
    )(x, ctx, Wo.reshape(_H * _DH, _D).astype(jnp.bfloat16),
      bo.reshape(1, _D), g1.reshape(1, _D), beta1.reshape(1, _D),
      W1.astype(jnp.bfloat16), bf1.reshape(1, _FF),
      W2.astype(jnp.bfloat16), bf2.reshape(1, _D),
      g2.reshape(1, _D), beta2.reshape(1, _D))

    return out[None]


# speedup vs baseline: 2.6410x; 2.1471x over previous
import functools

import jax
import jax.numpy as jnp
import numpy as np
from jax.experimental import pallas as pl

_S = 2048
_BS = 64
_NB = _S // _BS
_H = 12
_DH = 64
_D = 768
_FF = 3072
_NR = 3
_G = 3
_HG = _H // _G
_GW = _HG * _DH
_MAXC = 8


def _block_mask() -> np.ndarray:
    m = np.zeros((_NB, _NB), dtype=bool)
    idx = np.arange(_NB)
    for off in (-1, 0, 1):
        j = idx + off
        ok = (j >= 0) & (j < _NB)
        m[idx[ok], j[ok]] = True
    m[0, :] = True
    m[-1, :] = True
    m[:, 0] = True
    m[:, -1] = True
    rng = np.random.RandomState(0)
    for i in range(1, _NB - 1):
        cand = np.array([j for j in range(1, _NB - 1) if abs(j - i) > 1])
        sel = rng.choice(cand, size=_NR, replace=False)
        m[i, sel] = True
    return m


_BM = _block_mask()
_COLS = {}
_NVAL = {}
for _i in range(1, _NB - 1):
    _c = list(np.where(_BM[_i])[0])
    _NVAL[_i] = len(_c)
    while len(_c) < _MAXC:
        _c.append(_c[-1])
    _COLS[_i] = _c


def _attn_kernel(x_ref, wq_ref, wk_ref, wv_ref, bq_ref, bk_ref, bv_ref,
                 o_ref):
    x = x_ref[...]
    qf = (jnp.dot(x, wq_ref[...], preferred_element_type=jnp.float32)
          + bq_ref[...])
    kf = (jnp.dot(x, wk_ref[...], preferred_element_type=jnp.float32)
          + bk_ref[...])
    vf = (jnp.dot(x, wv_ref[...], preferred_element_type=jnp.float32)
          + bv_ref[...])
    kb = kf.astype(jnp.bfloat16)
    vb = vf.astype(jnp.bfloat16)

    lane_g = jax.lax.broadcasted_iota(jnp.int32, (_S, _GW), 1) // _DH
    qm = [(qf * jnp.where(lane_g == h, jnp.float32(0.125),
                          jnp.float32(0.0))).astype(jnp.bfloat16)
          for h in range(_HG)]
    out_lane = jax.lax.broadcasted_iota(jnp.int32, (_BS, _GW), 1) // _DH

    outs = []
    for i in range(_NB):
        if i == 0 or i == _NB - 1:
            kg, vg = kb, vb
            nval = _NB
        else:
            cols = _COLS[i]
            nval = _NVAL[i]
            kg = jnp.concatenate(
                [kb[c * _BS:(c + 1) * _BS, :] for c in cols], axis=0)
            vg = jnp.concatenate(
                [vb[c * _BS:(c + 1) * _BS, :] for c in cols], axis=0)
        nk = kg.shape[0]
        qs = jnp.concatenate(
            [qm[h][i * _BS:(i + 1) * _BS, :] for h in range(_HG)], axis=0)
        s = jax.lax.dot_general(
            qs, kg, (((1,), (1,)), ((), ())),
            preferred_element_type=jnp.float32)
        if nval * _BS < nk:
            lane = jax.lax.broadcasted_iota(jnp.int32, (_HG * _BS, nk), 1)
            s = jnp.where(lane < nval * _BS, s, jnp.float32(-1e9))
        m = jnp.max(s, axis=1, keepdims=True)
        e = jnp.exp(s - m)
        p = (e / jnp.sum(e, axis=1, keepdims=True)).astype(jnp.bfloat16)
        cs = jax.lax.dot_general(
            p, vg, (((1,), (0,)), ((), ())),
            preferred_element_type=jnp.float32)
        acc = jnp.zeros((_BS, _GW), jnp.float32)
        for h in range(_HG):
            acc = acc + jnp.where(
                out_lane == h, cs[h * _BS:(h + 1) * _BS, :], 0.0)
        outs.append(acc.astype(jnp.bfloat16))
    o_ref[...] = jnp.concatenate(outs, axis=0)


def _ln(t, g, b):
    mu = jnp.mean(t, axis=1, keepdims=True)
    var = jnp.mean((t - mu) ** 2, axis=1, keepdims=True)
    return (t - mu) / jnp.sqrt(var + 1e-6) * g + b


def _ffn_kernel(x_ref, c_ref, wo_ref, bo_ref, g1_ref, b1_ref, w1_ref,
                bf1_ref, w2_ref, bf2_ref, g2_ref, b2_ref, o_ref):
    attn = (
        jnp.dot(c_ref[...], wo_ref[...], preferred_element_type=jnp.float32)
        + bo_ref[...]
    )
    out1 = _ln(x_ref[...] + attn, g1_ref[...], b1_ref[...])
    h = jnp.maximum(
        jnp.dot(out1.astype(jnp.bfloat16), w1_ref[...],
                preferred_element_type=jnp.float32)
        + bf1_ref[...], 0.0)
    ffn = (
        jnp.dot(h.astype(jnp.bfloat16), w2_ref[...],
                preferred_element_type=jnp.float32)
        + bf2_ref[...]
    )
    o_ref[...] = _ln(out1 + ffn, g2_ref[...], b2_ref[...])


_ROWC = 256
_NRC = _S // _ROWC


@functools.partial(jax.jit, static_argnums=())
def kernel(inputs, Wq, bq, Wk, bk, Wv, bv, Wo, bo, g1, beta1, g2, beta2,
           W1, bf1, W2, bf2):
    x = inputs[0]
    xb = x.astype(jnp.bfloat16)
    wq = Wq.reshape(_D, _H * _DH).astype(jnp.bfloat16)
    wk = Wk.reshape(_D, _H * _DH).astype(jnp.bfloat16)
    wv = Wv.reshape(_D, _H * _DH).astype(jnp.bfloat16)
    bq2 = bq.reshape(1, _H * _DH)
    bk2 = bk.reshape(1, _H * _DH)
    bv2 = bv.reshape(1, _H * _DH)

    ctx = pl.pallas_call(
        _attn_kernel,
        grid=(_G,),
        in_specs=[
            pl.BlockSpec((_S, _D), lambda g: (0, 0)),
            pl.BlockSpec((_D, _GW), lambda g: (0, g)),
            pl.BlockSpec((_D, _GW), lambda g: (0, g)),
            pl.BlockSpec((_D, _GW), lambda g: (0, g)),
            pl.BlockSpec((1, _GW), lambda g: (0, g)),
            pl.BlockSpec((1, _GW), lambda g: (0, g)),
            pl.BlockSpec((1, _GW), lambda g: (0, g)),
        ],
        out_specs=pl.BlockSpec((_S, _GW), lambda g: (0, g)),
        out_shape=jax.ShapeDtypeStruct((_S, _H * _DH), jnp.bfloat16),
    )(xb, wq, wk, wv, bq2, bk2, bv2)

    out = pl.pallas_call(
        _ffn_kernel,
        grid=(_NRC,),
        in_specs=[
            pl.BlockSpec((_ROWC, _D), lambda r: (r, 0)),
            pl.BlockSpec((_ROWC, _H * _DH), lambda r: (r, 0)),
            pl.BlockSpec((_H * _DH, _D), lambda r: (0, 0)),
            pl.BlockSpec((1, _D), lambda r: (0, 0)),
            pl.BlockSpec((1, _D), lambda r: (0, 0)),
            pl.BlockSpec((1, _D), lambda r: (0, 0)),
            pl.BlockSpec((_D, _FF), lambda r: (0, 0)),
            pl.BlockSpec((1, _FF), lambda r: (0, 0)),
            pl.BlockSpec((_FF, _D), lambda r: (0, 0)),
            pl.BlockSpec((1, _D), lambda r: (0, 0)),
            pl.BlockSpec((1, _D), lambda r: (0, 0)),
            pl.BlockSpec((1, _D), lambda r: (0, 0)),
        ],
        out_specs=pl.BlockSpec((_ROWC, _D), lambda r: (r, 0)),
        out_shape=jax.ShapeDtypeStruct((_S, _D), jnp.float32),
    )(x, ctx, Wo.reshape(_H * _DH, _D).astype(jnp.bfloat16),
      bo.reshape(1, _D), g1.reshape(1, _D), beta1.reshape(1, _D),
      W1.astype(jnp.bfloat16), bf1.reshape(1, _FF),
      W2.astype(jnp.bfloat16), bf2.reshape(1, _D),
      g2.reshape(1, _D), beta2.reshape(1, _D))

    return out[None]
